# baseline (speedup 1.0000x reference)
import functools

import jax
import jax.numpy as jnp
from jax import lax
from jax.experimental import pallas as pl
from jax.experimental.pallas import tpu as pltpu
from jax.experimental.pallas import tpu_sc as plsc

N = 10000
E = 320000
D = 128
L = 16
NC = 2
NS = 16
NW = NC * NS
EPW = E // NW
C = 80
NCHUNK = EPW // C
N_PAD = 10240
RPT = N_PAD // NS
RCH = C
NRC = RPT // RCH
NBUF = 4
NIB = 8


def _elr_body(feat_ref, w_ref, b_ref, el_ref, er_ref):
    elr = (
        jnp.dot(feat_ref[...], w_ref[...], preferred_element_type=jnp.float32)
        + b_ref[...]
    )
    el_ref[...] = elr[:, 0]
    er_ref[...] = elr[:, 1]


def _combine_body(p_ref, o_ref):
    o_ref[...] = p_ref[0, :N] + p_ref[1, :N]


_sc_mesh = plsc.VectorSubcoreMesh(
    core_axis_name="c", subcore_axis_name="s", num_cores=NC, num_subcores=NS
)


@functools.partial(
    pl.kernel,
    out_type=jax.ShapeDtypeStruct((NC * N_PAD, D), jnp.float32),
    mesh=_sc_mesh,
    scratch_types=[
        [
            dict(
                elg=pltpu.VMEM((C,), jnp.float32),
                erg=pltpu.VMEM((C,), jnp.float32),
                a=pltpu.VMEM((C,), jnp.float32),
                rows=pltpu.VMEM((C, D), jnp.float32),
                gsem=pltpu.SemaphoreType.DMA,
                ssem=pltpu.SemaphoreType.DMA,
            )
            for _ in range(NBUF)
        ],
        [
            dict(
                src=pltpu.VMEM((C,), jnp.int32),
                dst=pltpu.VMEM((C,), jnp.int32),
                isem=pltpu.SemaphoreType.DMA,
            )
            for _ in range(NIB)
        ],
        pltpu.VMEM_SHARED((N_PAD, D), jnp.float32),
    ],
    compiler_params=pltpu.CompilerParams(needs_layout_passes=False),
)
def _sc_edges(ei_hbm, el_hbm, er_hbm, feat_hbm, out_hbm,
              rbufs, ibufs, acc):
    cid = lax.axis_index("c")
    sid = lax.axis_index("s")
    wid = sid * NC + cid
    ebase = wid * EPW

    def _fire_idx(ib, chunk):
        base = ebase + chunk * C
        pltpu.async_copy(ei_hbm.at[pl.ds(base, C)], ib["src"], ib["isem"])
        pltpu.async_copy(ei_hbm.at[pl.ds(E + base, C)], ib["dst"], ib["isem"])

    def _wait_idx(ib):
        pltpu.make_async_copy(ei_hbm.at[pl.ds(0, C)], ib["src"], ib["isem"]).wait()
        pltpu.make_async_copy(ei_hbm.at[pl.ds(0, C)], ib["dst"], ib["isem"]).wait()

    def _fire_gather(rb, ib):
        pltpu.async_copy(el_hbm.at[ib["src"]], rb["elg"], rb["gsem"])
        pltpu.async_copy(er_hbm.at[ib["dst"]], rb["erg"], rb["gsem"])
        pltpu.async_copy(feat_hbm.at[ib["src"]], rb["rows"], rb["gsem"])

    def _wait_gather_scalars(rb):
        pltpu.make_async_copy(el_hbm.at[pl.ds(0, C)], rb["elg"], rb["gsem"]).wait()
        pltpu.make_async_copy(er_hbm.at[pl.ds(0, C)], rb["erg"], rb["gsem"]).wait()

    def _wait_gather_rows(rb):
        pltpu.make_async_copy(feat_hbm.at[pl.ds(0, C)], rb["rows"], rb["gsem"]).wait()

    def _wait_scatter(rb):
        pltpu.make_async_copy(out_hbm.at[pl.ds(0, C)], rb["rows"], rb["ssem"]).wait()

    z = rbufs[0]["rows"]

    def _zrow(r, carry):
        for g in range(D // L):
            z[r, pl.ds(g * L, L)] = jnp.zeros((L,), jnp.float32)
        return carry

    lax.fori_loop(0, RCH, _zrow, 0)
    row0 = sid * RPT
    for j in range(NRC):
        pltpu.sync_copy(z, acc.at[pl.ds(row0 + j * RCH, RCH)])
    plsc.subcore_barrier()

    for j in range(4):
        _fire_idx(ibufs[j], j)
    for j in range(3):
        _wait_idx(ibufs[j])
        _fire_gather(rbufs[j], ibufs[j])

    def _slot(i, carry):
        m8 = lax.rem(i, jnp.int32(NIB))
        for s in range(NIB):
            b = s % NBUF

            @pl.when(m8 == s)
            def _case():
                rb = rbufs[b]
                ib = ibufs[s]

                @pl.when(i + 4 < NCHUNK)
                def _pf_idx():
                    _fire_idx(ibufs[(s + 4) % NIB], i + 4)

                @pl.when(i + 3 < NCHUNK)
                def _pf_rows():
                    @pl.when(i >= 1)
                    def _dr():
                        _wait_scatter(rbufs[(b + 3) % NBUF])

                    ib2 = ibufs[(s + 3) % NIB]
                    _wait_idx(ib2)
                    _fire_gather(rbufs[(b + 3) % NBUF], ib2)

                _wait_gather_scalars(rb)
                for g in range(C // L):
                    e = (rb["elg"][pl.ds(g * L, L)]
                         + rb["erg"][pl.ds(g * L, L)])
                    rb["a"][pl.ds(g * L, L)] = jnp.where(e > 0, e, 0.2 * e)

                _wait_gather_rows(rb)
                av = rb["a"]
                rv = rb["rows"]

                @plsc.parallel_loop(0, C, unroll=8)
                def _row(r):
                    bc = plsc.load_gather(av, [jnp.zeros((L,), jnp.int32) + r])
                    for g in range(D // L):
                        sl = (r, pl.ds(g * L, L))
                        rv[sl] = rv[sl] * bc

                pltpu.async_copy(rv, acc.at[ib["dst"]], rb["ssem"], add=True)

        return carry

    lax.fori_loop(0, NCHUNK, _slot, 0)
    for b in range(NBUF):
        _wait_scatter(rbufs[b])

    plsc.subcore_barrier()
    st = rbufs[0]["rows"]
    for j in range(NRC):
        r0 = sid * RPT + j * RCH
        pltpu.sync_copy(acc.at[pl.ds(r0, RCH)], st)
        pltpu.sync_copy(st, out_hbm.at[pl.ds(cid * N_PAD + r0, RCH)])


def kernel(feat, edge_index, wl, bl, wr, br):
    w2 = jnp.concatenate([wl, wr], axis=1)
    b2 = jnp.concatenate([bl, br]).reshape(1, 2)
    el, er = pl.pallas_call(
        _elr_body,
        out_shape=(
            jax.ShapeDtypeStruct((N,), jnp.float32),
            jax.ShapeDtypeStruct((N,), jnp.float32),
        ),
    )(feat, w2, b2)
    ei = edge_index.astype(jnp.int32).reshape(2 * E)
    parts = _sc_edges(ei, el, er, feat)
    out = pl.pallas_call(
        _combine_body,
        out_shape=jax.ShapeDtypeStruct((N, D), jnp.float32),
    )(parts.reshape(NC, N_PAD, D))
    return out


# speedup vs baseline: 1.0718x; 1.0718x over previous
import functools

import jax
import jax.numpy as jnp
from jax import lax
from jax.experimental import pallas as pl
from jax.experimental.pallas import tpu as pltpu
from jax.experimental.pallas import tpu_sc as plsc

N = 10000
E = 320000
D = 128
L = 16
NC = 2
NS = 16
NW = NC * NS
EPW = E // NW
C = 80
NCHUNK = EPW // C
N_PAD = 10240
RPT = N_PAD // NS
RCH = C
NRC = RPT // RCH
NBUF = 4
NIB = 8


def _elr_body(feat_ref, w_ref, b_ref, el_ref, er_ref):
    elr = (
        jnp.dot(feat_ref[...], w_ref[...], preferred_element_type=jnp.float32)
        + b_ref[...]
    )
    el_ref[...] = elr[:, 0]
    er_ref[...] = elr[:, 1]


def _combine_body(p_ref, o_ref):
    o_ref[...] = p_ref[0, :N] + p_ref[1, :N]


_sc_mesh = plsc.VectorSubcoreMesh(
    core_axis_name="c", subcore_axis_name="s", num_cores=NC, num_subcores=NS
)


@functools.partial(
    pl.kernel,
    out_type=jax.ShapeDtypeStruct((NC * N_PAD, D), jnp.float32),
    mesh=_sc_mesh,
    scratch_types=[
        [
            dict(
                elg=pltpu.VMEM((C,), jnp.float32),
                erg=pltpu.VMEM((C,), jnp.float32),
                a=pltpu.VMEM((C,), jnp.float32),
                rows=pltpu.VMEM((C, D), jnp.float32),
                gsem=pltpu.SemaphoreType.DMA,
                ssem=pltpu.SemaphoreType.DMA,
            )
            for _ in range(NBUF)
        ],
        [
            dict(
                src=pltpu.VMEM((C,), jnp.int32),
                dst=pltpu.VMEM((C,), jnp.int32),
                isem=pltpu.SemaphoreType.DMA,
            )
            for _ in range(NIB)
        ],
        pltpu.VMEM_SHARED((N_PAD, D), jnp.float32),
    ],
    compiler_params=pltpu.CompilerParams(needs_layout_passes=False),
)
def _sc_edges(ei_hbm, el_hbm, er_hbm, feat_hbm, out_hbm,
              rbufs, ibufs, acc):
    cid = lax.axis_index("c")
    sid = lax.axis_index("s")
    wid = sid * NC + cid
    ebase = wid * EPW

    def _fire_idx(ib, chunk):
        base = ebase + chunk * C
        pltpu.async_copy(ei_hbm.at[pl.ds(base, C)], ib["src"], ib["isem"])
        pltpu.async_copy(ei_hbm.at[pl.ds(E + base, C)], ib["dst"], ib["isem"])

    def _wait_idx(ib):
        pltpu.make_async_copy(ei_hbm.at[pl.ds(0, C)], ib["src"], ib["isem"]).wait()
        pltpu.make_async_copy(ei_hbm.at[pl.ds(0, C)], ib["dst"], ib["isem"]).wait()

    def _fire_gather(rb, ib):
        pltpu.async_copy(el_hbm.at[ib["src"]], rb["elg"], rb["gsem"])
        pltpu.async_copy(er_hbm.at[ib["dst"]], rb["erg"], rb["gsem"])
        h = C // 2
        pltpu.async_copy(feat_hbm.at[ib["src"].at[pl.ds(0, h)]],
                         rb["rows"].at[pl.ds(0, h)], rb["gsem"])
        pltpu.async_copy(feat_hbm.at[ib["src"].at[pl.ds(h, h)]],
                         rb["rows"].at[pl.ds(h, h)], rb["gsem"])

    def _wait_gather_scalars(rb):
        pltpu.make_async_copy(el_hbm.at[pl.ds(0, C)], rb["elg"], rb["gsem"]).wait()
        pltpu.make_async_copy(er_hbm.at[pl.ds(0, C)], rb["erg"], rb["gsem"]).wait()

    def _wait_gather_rows(rb):
        pltpu.make_async_copy(feat_hbm.at[pl.ds(0, C)], rb["rows"], rb["gsem"]).wait()


    def _wait_scatter(rb):
        pltpu.make_async_copy(out_hbm.at[pl.ds(0, C)], rb["rows"], rb["ssem"]).wait()

    z = rbufs[0]["rows"]

    def _zrow(r, carry):
        for g in range(D // L):
            z[r, pl.ds(g * L, L)] = jnp.zeros((L,), jnp.float32)
        return carry

    lax.fori_loop(0, RCH, _zrow, 0)
    row0 = sid * RPT
    for j in range(NRC):
        pltpu.sync_copy(z, acc.at[pl.ds(row0 + j * RCH, RCH)])
    plsc.subcore_barrier()

    for j in range(4):
        _fire_idx(ibufs[j], j)
    for j in range(2):
        _wait_idx(ibufs[j])
        _fire_gather(rbufs[j], ibufs[j])

    def _slot(i, carry):
        m8 = lax.rem(i, jnp.int32(NIB))
        for s in range(NIB):
            b = s % NBUF

            @pl.when(m8 == s)
            def _case():
                rb = rbufs[b]
                ib = ibufs[s]

                @pl.when(i + 4 < NCHUNK)
                def _pf_idx():
                    _fire_idx(ibufs[(s + 4) % NIB], i + 4)

                @pl.when(i + 2 < NCHUNK)
                def _pf_rows():
                    @pl.when(i >= 2)
                    def _dr():
                        _wait_scatter(rbufs[(b + 2) % NBUF])

                    ib2 = ibufs[(s + 2) % NIB]
                    _wait_idx(ib2)
                    _fire_gather(rbufs[(b + 2) % NBUF], ib2)

                _wait_gather_scalars(rb)
                for g in range(C // L):
                    e = (rb["elg"][pl.ds(g * L, L)]
                         + rb["erg"][pl.ds(g * L, L)])
                    rb["a"][pl.ds(g * L, L)] = jnp.where(e > 0, e, 0.2 * e)

                _wait_gather_rows(rb)
                av = rb["a"]
                rv = rb["rows"]

                @plsc.parallel_loop(0, C, unroll=8)
                def _row(r):
                    bc = plsc.load_gather(av, [jnp.zeros((L,), jnp.int32) + r])
                    for g in range(D // L):
                        sl = (r, pl.ds(g * L, L))
                        rv[sl] = rv[sl] * bc

                pltpu.async_copy(rv, acc.at[ib["dst"]], rb["ssem"], add=True)

        return carry

    lax.fori_loop(0, NCHUNK, _slot, 0)
    for b in range(NBUF):
        _wait_scatter(rbufs[b])

    plsc.subcore_barrier()
    st = rbufs[0]["rows"]
    for j in range(NRC):
        r0 = sid * RPT + j * RCH
        pltpu.sync_copy(acc.at[pl.ds(r0, RCH)], st)
        pltpu.sync_copy(st, out_hbm.at[pl.ds(cid * N_PAD + r0, RCH)])


def kernel(feat, edge_index, wl, bl, wr, br):
    w2 = jnp.concatenate([wl, wr], axis=1)
    b2 = jnp.concatenate([bl, br]).reshape(1, 2)
    el, er = pl.pallas_call(
        _elr_body,
        out_shape=(
            jax.ShapeDtypeStruct((N,), jnp.float32),
            jax.ShapeDtypeStruct((N,), jnp.float32),
        ),
    )(feat, w2, b2)
    ei = edge_index.astype(jnp.int32).reshape(2 * E)
    parts = _sc_edges(ei, el, er, feat)
    out = pl.pallas_call(
        _combine_body,
        out_shape=jax.ShapeDtypeStruct((N, D), jnp.float32),
    )(parts.reshape(NC, N_PAD, D))
    return out
